# baseline (speedup 1.0000x reference)
import functools

import jax
import jax.numpy as jnp
from jax import lax
from jax.experimental import pallas as pl
from jax.experimental.pallas import tpu as pltpu
from jax.experimental.pallas import tpu_sc as plsc

B = 16384
D = 64
NC = 2
NS = 16
NW = NC * NS
BPW = B // NW
CHUNK = 128
NCHUNK = BPW // CHUNK
PAIRW = 2 * D
TBLK = 512


def _tc_widen_rows(t_ref, out_ref):
    t = t_ref[...].T
    out_ref[...] = jnp.concatenate([t, t], axis=1)


def _to_row_major(table_t):
    nfeat, v = table_t.shape
    grid = (v + TBLK - 1) // TBLK
    return pl.pallas_call(
        _tc_widen_rows,
        grid=(grid,),
        in_specs=[pl.BlockSpec((nfeat, TBLK), lambda j: (0, j))],
        out_specs=pl.BlockSpec((TBLK, 2 * nfeat), lambda j: (j, 0)),
        out_shape=jax.ShapeDtypeStruct((v, 2 * nfeat), jnp.float32),
    )(table_t)


def _sc_cf_dot(x_hbm, user_hbm, item_hbm, out_hbm,
               x_v, idx_u, idx_i, buf_u, buf_i, out_v, sem_u, sem_i):
    cid = lax.axis_index("c")
    sid = lax.axis_index("s")
    wid = sid * NC + cid
    base = wid * BPW

    pltpu.sync_copy(x_hbm.at[pl.ds(base * 2, BPW * 2)], x_v)

    iota16 = lax.iota(jnp.int32, 16)
    for g in range(BPW // 16):
        even16 = (iota16 + (g * 16)) * 2
        c = (g * 16) // CHUNK
        off = (g * 16) % CHUNK
        idx_u[c, pl.ds(off, 16)] = plsc.load_gather(x_v, [even16])
        idx_i[c, pl.ds(off, 16)] = plsc.load_gather(x_v, [even16 + 1])

    def fire(c, slot):
        cu = pltpu.async_copy(user_hbm.at[idx_u.at[c]], buf_u.at[slot], sem_u)
        ci = pltpu.async_copy(item_hbm.at[idx_i.at[c]], buf_i.at[slot], sem_i)
        return cu, ci

    lane15 = iota16 == 15
    inflight = fire(0, 0)
    for c in range(NCHUNK):
        if c + 1 < NCHUNK:
            nxt = fire(c + 1, (c + 1) % 2)
        for cp in inflight:
            cp.wait()
        slot = c % 2

        def body(rr, carry, c=c, slot=slot):
            acc = (buf_u[slot, rr, pl.ds(0, 16)]
                   * buf_i[slot, rr, pl.ds(0, 16)])
            for k in range(1, D // 16):
                acc = acc + (buf_u[slot, rr, pl.ds(k * 16, 16)]
                             * buf_i[slot, rr, pl.ds(k * 16, 16)])
            cs = plsc.cumsum(acc)
            plsc.store_scatter(
                out_v, [jnp.full((16,), c * CHUNK, jnp.int32) + rr], cs,
                mask=lane15)
            return carry
        lax.fori_loop(0, CHUNK, body, 0)
        if c + 1 < NCHUNK:
            inflight = nxt

    pltpu.sync_copy(out_v, out_hbm.at[pl.ds(base, BPW)])


@jax.jit
def kernel(x, user_emb, item_emb):
    mesh = plsc.VectorSubcoreMesh(core_axis_name="c", subcore_axis_name="s")
    f = functools.partial(
        pl.kernel,
        mesh=mesh,
        out_type=jax.ShapeDtypeStruct((B,), jnp.float32),
        scratch_types=[
            pltpu.VMEM((BPW * 2,), jnp.int32),
            pltpu.VMEM((NCHUNK, CHUNK), jnp.int32),
            pltpu.VMEM((NCHUNK, CHUNK), jnp.int32),
            pltpu.VMEM((2, CHUNK, PAIRW), jnp.float32),
            pltpu.VMEM((2, CHUNK, PAIRW), jnp.float32),
            pltpu.VMEM((BPW,), jnp.float32),
            pltpu.SemaphoreType.DMA,
            pltpu.SemaphoreType.DMA,
        ],
        compiler_params=pltpu.CompilerParams(
            needs_layout_passes=False, use_tc_tiling_on_sc=True),
    )(_sc_cf_dot)
    return f(x.astype(jnp.int32).reshape(-1),
             _to_row_major(user_emb.T),
             _to_row_major(item_emb.T))


# speedup vs baseline: 1.0243x; 1.0243x over previous
import functools

import jax
import jax.numpy as jnp
from jax import lax
from jax.experimental import pallas as pl
from jax.experimental.pallas import tpu as pltpu
from jax.experimental.pallas import tpu_sc as plsc

B = 16384
D = 64
NC = 2
NS = 16
NW = NC * NS
BPW = B // NW
CHUNK = 128
NCHUNK = BPW // CHUNK
PAIRW = 2 * D
TBLK = 512


def _tc_widen_rows(t_ref, out_ref):
    a = t_ref[...]
    eye = jnp.eye(D, dtype=jnp.float32)
    t = lax.dot_general(a, eye, (((0,), (0,)), ((), ())),
                        preferred_element_type=jnp.float32)
    out_ref[:, 0:D] = t


def _to_row_major(table_t):
    nfeat, v = table_t.shape
    grid = (v + TBLK - 1) // TBLK
    return pl.pallas_call(
        _tc_widen_rows,
        grid=(grid,),
        in_specs=[pl.BlockSpec((nfeat, TBLK), lambda j: (0, j))],
        out_specs=pl.BlockSpec((TBLK, 2 * nfeat), lambda j: (j, 0)),
        out_shape=jax.ShapeDtypeStruct((v, 2 * nfeat), jnp.float32),
    )(table_t)


def _sc_cf_dot(xu_hbm, xi_hbm, user_hbm, item_hbm, out_hbm,
               idx_u, idx_i, buf_u, buf_i, out_v, sem_u, sem_i):
    cid = lax.axis_index("c")
    sid = lax.axis_index("s")
    wid = sid * NC + cid
    base = wid * BPW

    pltpu.sync_copy(xu_hbm.at[pl.ds(base, BPW)], idx_u)
    pltpu.sync_copy(xi_hbm.at[pl.ds(base, BPW)], idx_i)

    def fire(c, slot):
        cu = pltpu.async_copy(
            user_hbm.at[idx_u.at[pl.ds(c * CHUNK, CHUNK)]],
            buf_u.at[slot], sem_u)
        ci = pltpu.async_copy(
            item_hbm.at[idx_i.at[pl.ds(c * CHUNK, CHUNK)]],
            buf_i.at[slot], sem_i)
        return cu, ci

    iota16 = lax.iota(jnp.int32, 16)
    lane15 = iota16 == 15
    inflight = fire(0, 0)
    for c in range(NCHUNK):
        if c + 1 < NCHUNK:
            nxt = fire(c + 1, (c + 1) % 2)
        for cp in inflight:
            cp.wait()
        slot = c % 2

        def body(rr, carry, c=c, slot=slot):
            acc = (buf_u[slot, rr, pl.ds(0, 16)]
                   * buf_i[slot, rr, pl.ds(0, 16)])
            for k in range(1, D // 16):
                acc = acc + (buf_u[slot, rr, pl.ds(k * 16, 16)]
                             * buf_i[slot, rr, pl.ds(k * 16, 16)])
            cs = plsc.cumsum(acc)
            plsc.store_scatter(
                out_v, [jnp.full((16,), c * CHUNK, jnp.int32) + rr], cs,
                mask=lane15)
            return carry
        lax.fori_loop(0, CHUNK, body, 0)
        if c + 1 < NCHUNK:
            inflight = nxt

    pltpu.sync_copy(out_v, out_hbm.at[pl.ds(base, BPW)])


@jax.jit
def kernel(x, user_emb, item_emb):
    mesh = plsc.VectorSubcoreMesh(core_axis_name="c", subcore_axis_name="s")
    f = functools.partial(
        pl.kernel,
        mesh=mesh,
        out_type=jax.ShapeDtypeStruct((B,), jnp.float32),
        scratch_types=[
            pltpu.VMEM((BPW,), jnp.int32),
            pltpu.VMEM((BPW,), jnp.int32),
            pltpu.VMEM((2, CHUNK, PAIRW), jnp.float32),
            pltpu.VMEM((2, CHUNK, PAIRW), jnp.float32),
            pltpu.VMEM((BPW,), jnp.float32),
            pltpu.SemaphoreType.DMA,
            pltpu.SemaphoreType.DMA,
        ],
        compiler_params=pltpu.CompilerParams(
            needs_layout_passes=False, use_tc_tiling_on_sc=True),
    )(_sc_cf_dot)
    x32 = x.astype(jnp.int32)
    return f(x32[:, 0], x32[:, 1],
             _to_row_major(user_emb.T),
             _to_row_major(item_emb.T))
